# baseline (speedup 1.0000x reference)
import functools

import jax
import jax.numpy as jnp
from jax import lax
from jax.experimental import pallas as pl
from jax.experimental.pallas import tpu as pltpu
from jax.experimental.pallas import tpu_sc as plsc

_WAVELENGTH_MIN = 0.01
_WAVELENGTH_MAX = 0.5
_FOCAL_DEPTH = 0.5

_L = 16
_NC = 2
_NS = 16
_NW = _NC * _NS
_COLS = 512
_ROWS = 64 * 512
_ROWS_W = _ROWS // _NW
_CHUNK_R = 32
_NCH = _ROWS_W // _CHUNK_R
_NBUF = 4


def _body(depth_hbm, w_hbm, out_hbm, wv, b0, b1, b2, b3,
          i0, i1, i2, i3, o0, o1, o2, o3):
    c = lax.axis_index("c")
    s = lax.axis_index("s")
    wid = s * _NC + c
    base = wid * _ROWS_W

    pltpu.sync_copy(w_hbm, wv)
    lam = jnp.clip(jnp.abs(wv[...]), _WAVELENGTH_MIN, _WAVELENGTH_MAX)
    scale = (2.0 * jnp.pi) / lam

    bufs = (b0, b1, b2, b3)
    isems = (i0, i1, i2, i3)
    osems = (o0, o1, o2, o3)

    def in_cp(k, b):
        return pltpu.make_async_copy(
            depth_hbm.at[pl.ds(base + k * _CHUNK_R, _CHUNK_R), :],
            bufs[b], isems[b])

    def out_cp(k, b):
        return pltpu.make_async_copy(
            bufs[b], out_hbm.at[pl.ds(base + k * _CHUNK_R, _CHUNK_R), :],
            osems[b])

    in_cp(0, 0).start()
    in_cp(1, 1).start()

    def step(t, carry):
        for b in range(_NBUF):
            k = _NBUF * t + b
            in_cp(k, b).wait()

            buf = bufs[b]

            @plsc.parallel_loop(0, _CHUNK_R, unroll=2)
            def _(r):
                for j in range(_COLS // _L):
                    x = buf[r, pl.ds(j * _L, _L)]
                    buf[r, pl.ds(j * _L, _L)] = scale * jnp.abs(x - _FOCAL_DEPTH)

            out_cp(k, b).start()

            nb = (b + 2) % _NBUF
            if b < 2:
                @pl.when(t > 0)
                def _():
                    out_cp(k - 2, nb).wait()

                in_cp(k + 2, nb).start()
            else:
                out_cp(k - 2, nb).wait()

                @pl.when(t + 1 < _NCH // _NBUF)
                def _():
                    in_cp(k + 2, nb).start()
        return carry

    lax.fori_loop(0, _NCH // _NBUF, step, 0)

    out_cp(_NCH - 2, (_NCH - 2) % _NBUF).wait()
    out_cp(_NCH - 1, (_NCH - 1) % _NBUF).wait()


@functools.partial(jax.jit, static_argnames=())
def kernel(depth, w_raw):
    w16 = jnp.broadcast_to(jnp.asarray(w_raw, jnp.float32), (_L,))
    rows = depth.reshape(_ROWS, _COLS)
    mesh = plsc.VectorSubcoreMesh(core_axis_name="c", subcore_axis_name="s")
    run = pl.kernel(
        _body,
        out_type=jax.ShapeDtypeStruct((_ROWS, _COLS), jnp.float32),
        mesh=mesh,
        compiler_params=pltpu.CompilerParams(use_tc_tiling_on_sc=True),
        scratch_types=[
            pltpu.VMEM((_L,), jnp.float32),
            pltpu.VMEM((_CHUNK_R, _COLS), jnp.float32),
            pltpu.VMEM((_CHUNK_R, _COLS), jnp.float32),
            pltpu.VMEM((_CHUNK_R, _COLS), jnp.float32),
            pltpu.VMEM((_CHUNK_R, _COLS), jnp.float32),
            pltpu.SemaphoreType.DMA,
            pltpu.SemaphoreType.DMA,
            pltpu.SemaphoreType.DMA,
            pltpu.SemaphoreType.DMA,
            pltpu.SemaphoreType.DMA,
            pltpu.SemaphoreType.DMA,
            pltpu.SemaphoreType.DMA,
            pltpu.SemaphoreType.DMA,
        ],
    )
    out = run(rows, w16)
    return out.reshape(depth.shape)


# speedup vs baseline: 1.0578x; 1.0578x over previous
import functools

import jax
import jax.numpy as jnp
from jax import lax
from jax.experimental import pallas as pl
from jax.experimental.pallas import tpu as pltpu
from jax.experimental.pallas import tpu_sc as plsc

_WAVELENGTH_MIN = 0.01
_WAVELENGTH_MAX = 0.5
_FOCAL_DEPTH = 0.5

_L = 16
_NC = 2
_NS = 16
_NW = _NC * _NS
_COLS = 512
_ROWS = 64 * 512
_ROWS_W = _ROWS // _NW
_CHUNK_R = 32
_NCH = _ROWS_W // _CHUNK_R
_NBUF = 4


def _body(depth_hbm, w_hbm, out_hbm, wv, ib0, ib1, ob0, ob1, is0, is1, os0, os1):
    c = lax.axis_index("c")
    s = lax.axis_index("s")
    wid = s * _NC + c
    base = wid * _ROWS_W

    pltpu.sync_copy(w_hbm, wv)
    lam = jnp.clip(jnp.abs(wv[...]), _WAVELENGTH_MIN, _WAVELENGTH_MAX)
    scale = (2.0 * jnp.pi) / lam

    ibs = (ib0, ib1)
    obs = (ob0, ob1)
    isems = (is0, is1)
    osems = (os0, os1)

    def in_cp(k, b):
        return pltpu.make_async_copy(
            depth_hbm.at[pl.ds(base + k * _CHUNK_R, _CHUNK_R), :],
            ibs[b], isems[b])

    def out_cp(k, b):
        return pltpu.make_async_copy(
            obs[b], out_hbm.at[pl.ds(base + k * _CHUNK_R, _CHUNK_R), :],
            osems[b])

    in_cp(0, 0).start()
    in_cp(1, 1).start()

    def step(t, carry):
        for b in range(2):
            k = 2 * t + b
            in_cp(k, b).wait()

            @pl.when(t > 0)
            def _():
                out_cp(k - 2, b).wait()

            ib = ibs[b]
            ob = obs[b]

            @plsc.parallel_loop(0, _CHUNK_R, unroll=2)
            def _(r):
                for j in range(_COLS // _L):
                    x = ib[r, pl.ds(j * _L, _L)]
                    ob[r, pl.ds(j * _L, _L)] = scale * jnp.abs(x - _FOCAL_DEPTH)

            out_cp(k, b).start()

            @pl.when(t + 1 < _NCH // 2)
            def _():
                in_cp(k + 2, b).start()
        return carry

    lax.fori_loop(0, _NCH // 2, step, 0)

    out_cp(_NCH - 2, 0).wait()
    out_cp(_NCH - 1, 1).wait()


@functools.partial(jax.jit, static_argnames=())
def kernel(depth, w_raw):
    w16 = jnp.broadcast_to(jnp.asarray(w_raw, jnp.float32), (_L,))
    rows = depth.reshape(_ROWS, _COLS)
    mesh = plsc.VectorSubcoreMesh(core_axis_name="c", subcore_axis_name="s")
    run = pl.kernel(
        _body,
        out_type=jax.ShapeDtypeStruct((_ROWS, _COLS), jnp.float32),
        mesh=mesh,
        compiler_params=pltpu.CompilerParams(
            use_tc_tiling_on_sc=True, skip_device_barrier=True),
        scratch_types=[
            pltpu.VMEM((_L,), jnp.float32),
            pltpu.VMEM((_CHUNK_R, _COLS), jnp.float32),
            pltpu.VMEM((_CHUNK_R, _COLS), jnp.float32),
            pltpu.VMEM((_CHUNK_R, _COLS), jnp.float32),
            pltpu.VMEM((_CHUNK_R, _COLS), jnp.float32),
            pltpu.SemaphoreType.DMA,
            pltpu.SemaphoreType.DMA,
            pltpu.SemaphoreType.DMA,
            pltpu.SemaphoreType.DMA,
        ],
    )
    out = run(rows, w16)
    return out.reshape(depth.shape)
